# baseline (speedup 1.0000x reference)
import functools

import jax
import jax.numpy as jnp
from jax import lax
from jax.experimental import pallas as pl
from jax.experimental.pallas import tpu as pltpu
from jax.experimental.pallas import tpu_sc as plsc

DEPTH = 5
PAD = 64

NUM_CORES = 2
NUM_SUBCORES = 16
LANES = 16
NUM_WORKERS = NUM_CORES * NUM_SUBCORES


def _make_kernel(batch, n_feat):
    rows_per_worker = batch // NUM_WORKERS
    vregs_per_worker = rows_per_worker // LANES

    mesh = plsc.VectorSubcoreMesh(
        core_axis_name="c",
        subcore_axis_name="s",
        num_cores=NUM_CORES,
        num_subcores=NUM_SUBCORES,
    )

    @functools.partial(
        pl.kernel,
        out_type=jax.ShapeDtypeStruct((batch,), jnp.float32),
        mesh=mesh,
        compiler_params=pltpu.CompilerParams(needs_layout_passes=False),
        scratch_types=[
            pltpu.VMEM((5 * PAD,), jnp.int32),
            pltpu.VMEM((rows_per_worker * n_feat,), jnp.float32),
            pltpu.VMEM((rows_per_worker,), jnp.float32),
            pltpu.SemaphoreType.DMA,
        ],
    )
    def tree_kernel(tab_hbm, x_hbm, out_hbm, tab_v, x_v, out_v, sem):
        wid = lax.axis_index("s") * NUM_CORES + lax.axis_index("c")
        base = wid * rows_per_worker

        copies = [
            pltpu.async_copy(tab_hbm, tab_v, sem),
            pltpu.async_copy(
                x_hbm.at[pl.ds(base * n_feat, rows_per_worker * n_feat)],
                x_v,
                sem,
            ),
        ]
        for c in copies:
            c.wait()

        lanes = lax.iota(jnp.int32, LANES)
        for j in range(vregs_per_worker):
            rows = lanes + (j * LANES)
            node = lanes * 0
            for _ in range(DEPTH):
                cl = plsc.load_gather(tab_v, [node + PAD])
                cr = plsc.load_gather(tab_v, [node + 2 * PAD])
                f = plsc.load_gather(tab_v, [node + 3 * PAD])
                thr = plsc.bitcast(
                    plsc.load_gather(tab_v, [node + 4 * PAD]), jnp.float32
                )
                fc = jnp.clip(f, 0, n_feat - 1)
                xf = plsc.load_gather(x_v, [rows * n_feat + fc])
                is_leaf = (cl == -1) & (cr == -1)
                nxt = jnp.where(xf <= thr, cl, cr)
                node = jnp.where(is_leaf, node, nxt)
            out_v[pl.ds(j * LANES, LANES)] = plsc.bitcast(
                plsc.load_gather(tab_v, [node]), jnp.float32
            )
        pltpu.sync_copy(out_v, out_hbm.at[pl.ds(base, rows_per_worker)])

    return tree_kernel


def kernel(X, children_left, children_right, feature, threshold, value):
    batch, n_feat = X.shape
    n_nodes = children_left.shape[0]
    pad = PAD - n_nodes

    def pad_i32(a):
        return jnp.pad(a.astype(jnp.int32), (0, pad))

    tab = jnp.concatenate(
        [
            pad_i32(lax.bitcast_convert_type(value.reshape(-1), jnp.int32)),
            pad_i32(children_left),
            pad_i32(children_right),
            pad_i32(feature),
            pad_i32(lax.bitcast_convert_type(threshold, jnp.int32)),
        ]
    )
    out = _make_kernel(batch, n_feat)(tab, X.reshape(-1))
    return out.reshape(batch, 1)


# speedup vs baseline: 1.1564x; 1.1564x over previous
import functools

import jax
import jax.numpy as jnp
from jax import lax
from jax.experimental import pallas as pl
from jax.experimental.pallas import tpu as pltpu
from jax.experimental.pallas import tpu_sc as plsc

DEPTH = 5
PAD = 64

NUM_CORES = 2
NUM_SUBCORES = 16
LANES = 16
NUM_WORKERS = NUM_CORES * NUM_SUBCORES


def _make_kernel(batch, n_feat, n_nodes):
    rows_per_worker = batch // NUM_WORKERS
    vregs_per_worker = rows_per_worker // LANES

    mesh = plsc.VectorSubcoreMesh(
        core_axis_name="c",
        subcore_axis_name="s",
        num_cores=NUM_CORES,
        num_subcores=NUM_SUBCORES,
    )

    @functools.partial(
        pl.kernel,
        out_type=jax.ShapeDtypeStruct((batch,), jnp.float32),
        mesh=mesh,
        compiler_params=pltpu.CompilerParams(needs_layout_passes=False),
        scratch_types=[
            pltpu.VMEM((4 * PAD,), jnp.int32),
            pltpu.VMEM((2 * PAD,), jnp.float32),
            pltpu.VMEM((rows_per_worker * n_feat,), jnp.float32),
            pltpu.VMEM((rows_per_worker,), jnp.float32),
            pltpu.SemaphoreType.DMA,
            pltpu.SemaphoreType.DMA,
        ],
    )
    def tree_kernel(cl_hbm, cr_hbm, f_hbm, thr_hbm, val_hbm, x_hbm, out_hbm,
                    tab_i, tab_f, x_v, out_v, sem_x, sem_tab):
        wid = lax.axis_index("s") * NUM_CORES + lax.axis_index("c")
        base = wid * rows_per_worker

        x_copy = pltpu.async_copy(
            x_hbm.at[pl.ds(base * n_feat, rows_per_worker * n_feat)],
            x_v,
            sem_x,
        )
        tab_copies = [
            pltpu.async_copy(cl_hbm, tab_i.at[pl.ds(PAD, n_nodes)], sem_tab),
            pltpu.async_copy(cr_hbm, tab_i.at[pl.ds(2 * PAD, n_nodes)], sem_tab),
            pltpu.async_copy(f_hbm, tab_i.at[pl.ds(3 * PAD, n_nodes)], sem_tab),
            pltpu.async_copy(val_hbm, tab_f.at[pl.ds(0, n_nodes)], sem_tab),
            pltpu.async_copy(thr_hbm, tab_f.at[pl.ds(PAD, n_nodes)], sem_tab),
        ]
        for c in tab_copies:
            c.wait()
        x_copy.wait()

        lanes = lax.iota(jnp.int32, LANES)
        for j in range(vregs_per_worker):
            rows = lanes + (j * LANES)
            node = lanes * 0
            for _ in range(DEPTH):
                cl = plsc.load_gather(tab_i, [node + PAD])
                cr = plsc.load_gather(tab_i, [node + 2 * PAD])
                f = plsc.load_gather(tab_i, [node + 3 * PAD])
                thr = plsc.load_gather(tab_f, [node + PAD])
                fc = jnp.clip(f, 0, n_feat - 1)
                xf = plsc.load_gather(x_v, [rows * n_feat + fc])
                is_leaf = (cl == -1) & (cr == -1)
                nxt = jnp.where(xf <= thr, cl, cr)
                node = jnp.where(is_leaf, node, nxt)
            out_v[pl.ds(j * LANES, LANES)] = plsc.load_gather(tab_f, [node])
        pltpu.sync_copy(out_v, out_hbm.at[pl.ds(base, rows_per_worker)])

    return tree_kernel


def kernel(X, children_left, children_right, feature, threshold, value):
    batch, n_feat = X.shape
    n_nodes = children_left.shape[0]
    out = _make_kernel(batch, n_feat, n_nodes)(
        children_left,
        children_right,
        feature,
        threshold,
        value.reshape(-1),
        X.reshape(-1),
    )
    return out.reshape(batch, 1)


# speedup vs baseline: 1.2140x; 1.0498x over previous
import functools

import jax
import jax.numpy as jnp
from jax import lax
from jax.experimental import pallas as pl
from jax.experimental.pallas import tpu as pltpu
from jax.experimental.pallas import tpu_sc as plsc

DEPTH = 5
PAD = 64

NUM_CORES = 1
NUM_SUBCORES = 16
LANES = 16
NUM_WORKERS = NUM_CORES * NUM_SUBCORES


def _make_kernel(batch, n_feat, n_nodes):
    rows_per_worker = batch // NUM_WORKERS
    vregs_per_worker = rows_per_worker // LANES

    mesh = plsc.VectorSubcoreMesh(
        core_axis_name="c",
        subcore_axis_name="s",
        num_cores=NUM_CORES,
        num_subcores=NUM_SUBCORES,
    )

    @functools.partial(
        pl.kernel,
        out_type=jax.ShapeDtypeStruct((batch,), jnp.float32),
        mesh=mesh,
        compiler_params=pltpu.CompilerParams(needs_layout_passes=False),
        scratch_types=[
            pltpu.VMEM((4 * PAD,), jnp.int32),
            pltpu.VMEM((2 * PAD,), jnp.float32),
            pltpu.VMEM((rows_per_worker * n_feat,), jnp.float32),
            pltpu.VMEM((rows_per_worker,), jnp.float32),
            pltpu.SemaphoreType.DMA,
            pltpu.SemaphoreType.DMA,
        ],
    )
    def tree_kernel(cl_hbm, cr_hbm, f_hbm, thr_hbm, val_hbm, x_hbm, out_hbm,
                    tab_i, tab_f, x_v, out_v, sem_x, sem_tab):
        wid = lax.axis_index("s") * NUM_CORES + lax.axis_index("c")
        base = wid * rows_per_worker

        x_copy = pltpu.async_copy(
            x_hbm.at[pl.ds(base * n_feat, rows_per_worker * n_feat)],
            x_v,
            sem_x,
        )
        tab_copies = [
            pltpu.async_copy(cl_hbm, tab_i.at[pl.ds(PAD, n_nodes)], sem_tab),
            pltpu.async_copy(cr_hbm, tab_i.at[pl.ds(2 * PAD, n_nodes)], sem_tab),
            pltpu.async_copy(f_hbm, tab_i.at[pl.ds(3 * PAD, n_nodes)], sem_tab),
            pltpu.async_copy(val_hbm, tab_f.at[pl.ds(0, n_nodes)], sem_tab),
            pltpu.async_copy(thr_hbm, tab_f.at[pl.ds(PAD, n_nodes)], sem_tab),
        ]
        for c in tab_copies:
            c.wait()
        x_copy.wait()

        lanes = lax.iota(jnp.int32, LANES)
        for j in range(vregs_per_worker):
            rows = lanes + (j * LANES)
            node = lanes * 0
            for _ in range(DEPTH):
                cl = plsc.load_gather(tab_i, [node + PAD])
                cr = plsc.load_gather(tab_i, [node + 2 * PAD])
                f = plsc.load_gather(tab_i, [node + 3 * PAD])
                thr = plsc.load_gather(tab_f, [node + PAD])
                fc = jnp.clip(f, 0, n_feat - 1)
                xf = plsc.load_gather(x_v, [rows * n_feat + fc])
                is_leaf = (cl == -1) & (cr == -1)
                nxt = jnp.where(xf <= thr, cl, cr)
                node = jnp.where(is_leaf, node, nxt)
            out_v[pl.ds(j * LANES, LANES)] = plsc.load_gather(tab_f, [node])
        pltpu.sync_copy(out_v, out_hbm.at[pl.ds(base, rows_per_worker)])

    return tree_kernel


def kernel(X, children_left, children_right, feature, threshold, value):
    batch, n_feat = X.shape
    n_nodes = children_left.shape[0]
    out = _make_kernel(batch, n_feat, n_nodes)(
        children_left,
        children_right,
        feature,
        threshold,
        value.reshape(-1),
        X.reshape(-1),
    )
    return out.reshape(batch, 1)
